# baseline (speedup 1.0000x reference)
import functools

import jax
import jax.numpy as jnp
from jax import lax
from jax.experimental import pallas as pl
from jax.experimental.pallas import tpu as pltpu
from jax.experimental.pallas import tpu_sc as plsc

_BS, _K, _N, _T = 64, 16, 32768, 3
_L = 16
_NW = 32
_BPW = _BS // _NW
_GROUP = 32
_CH = 32768
_NCH = _K * _N // _CH
_NG = _CH // (_GROUP * _L)
_NEG = float("-inf")


def _iota():
    return lax.iota(jnp.int32, _L)


def _sc_topk(joint_flat, tokens_flat):
    mesh = plsc.VectorSubcoreMesh(core_axis_name="c", subcore_axis_name="s")

    @functools.partial(
        pl.kernel,
        out_type=(
            jax.ShapeDtypeStruct((_BS * _K * (_T + 1),), jnp.int32),
            jax.ShapeDtypeStruct((_BS * _K,), jnp.float32),
        ),
        mesh=mesh,
        compiler_params=pltpu.CompilerParams(needs_layout_passes=False),
        scratch_types=[
            pltpu.VMEM((2 * _CH,), jnp.float32),
            pltpu.VMEM((_GROUP * _L + _L,), jnp.float32),
            pltpu.VMEM((_GROUP * _L + _L,), jnp.int32),
            pltpu.VMEM((_K * _T,), jnp.int32),
            pltpu.VMEM((_K * (_T + 1),), jnp.int32),
            pltpu.VMEM((_K,), jnp.float32),
            pltpu.SemaphoreType.DMA,
        ],
    )
    def run(jt_hbm, tok_hbm, otok_hbm, osc_hbm, lbuf, cbuf, ibuf, tbuf,
            obuf_t, obuf_s, sem):
        wid = lax.axis_index("s") * 2 + lax.axis_index("c")

        def batch_body(bi, _):
            b = wid * _BPW + bi
            base = b * _K * _N
            pltpu.sync_copy(tok_hbm.at[pl.ds(b * _K * _T, _K * _T)], tbuf)
            pltpu.async_copy(jt_hbm.at[pl.ds(base, _CH)],
                             lbuf.at[pl.ds(0, _CH)], sem)

            def chunk_body(ch, carry):
                cur = ch & 1
                loff = cur * _CH
                pltpu.make_async_copy(
                    jt_hbm.at[pl.ds(base + ch * _CH, _CH)],
                    lbuf.at[pl.ds(loff, _CH)], sem).wait()

                @pl.when(ch + 1 < _NCH)
                def _():
                    nxt = (ch + 1) & 1
                    pltpu.async_copy(
                        jt_hbm.at[pl.ds(base + (ch + 1) * _CH, _CH)],
                        lbuf.at[pl.ds(nxt * _CH, _CH)], sem)

                def group_body(g, carry2):
                    bv, bix, thr = carry2
                    p0 = g * (_GROUP * _L)
                    accm = lbuf[pl.ds(loff + p0, _L)]
                    for j in range(1, _GROUP):
                        accm = jnp.maximum(
                            accm, lbuf[pl.ds(loff + p0 + j * _L, _L)])

                    def slow(carry3):
                        thr3 = carry3[2]
                        offv = jnp.zeros((_L,), jnp.int32)
                        for j in range(_GROUP):
                            p = p0 + j * _L
                            c = lbuf[pl.ds(loff + p, _L)]
                            m = c > thr3
                            pos = plsc.cumsum(
                                jnp.where(m, 1, 0).astype(jnp.int32)) - 1
                            ov = ch * _CH + p + _iota()
                            plsc.store_scatter(cbuf, [offv + pos], c, mask=m)
                            plsc.store_scatter(ibuf, [offv + pos], ov, mask=m)
                            offv = offv + plsc.all_reduce_population_count(m)
                        plsc.store_scatter(
                            cbuf, [offv + _iota()],
                            jnp.full((_L,), _NEG, jnp.float32))
                        cnt = jnp.max(offv)
                        nmerge = (cnt + _L - 1) // _L

                        def merge_body(t, carry4):
                            bv5, bi5, _ = carry4
                            cv = cbuf[pl.ds(t * _L, _L)]
                            iv = ibuf[pl.ds(t * _L, _L)]
                            tr = lax.shift_right_logical(iv, 18)
                            ob = iv & (_K * _N // 2 - 1)
                            tc = lax.shift_right_logical(ob, 10)
                            r8 = lax.shift_right_logical(ob, 7) & 7
                            cc = ob & 127
                            idxv = (((tr << 3) + r8) << 15) + (tc << 7) + cc
                            sk, si = plsc.sort_key_val(
                                cv, idxv, descending=False)
                            rs = lax.rev(sk, (0,))
                            ri = lax.rev(si, (0,))
                            ge = bv5 >= rs
                            nv = jnp.where(ge, bv5, rs)
                            ni = jnp.where(ge, bi5, ri)
                            nv, ni = plsc.sort_key_val(
                                nv, ni, descending=False)
                            nthr = jnp.broadcast_to(jnp.min(nv), (_L,))
                            return (nv, ni, nthr)
                        return lax.fori_loop(0, nmerge, merge_body, carry3)

                    return lax.cond(jnp.any(accm > thr), slow,
                                    lambda c3: c3, (bv, bix, thr))
                return lax.fori_loop(0, _NG, group_body, carry)

            init = (jnp.full((_L,), _NEG, jnp.float32),
                    jnp.zeros((_L,), jnp.int32),
                    jnp.full((_L,), _NEG, jnp.float32))
            best_v, best_i, _ = lax.fori_loop(0, _NCH, chunk_body, init)

            sd = lax.rev(best_v, (0,))
            fd = lax.rev(best_i, (0,))
            beam = lax.shift_right_logical(fd, 15)
            newtok = fd & (_N - 1)
            lanes = _iota()
            for t in range(_T):
                gt = plsc.load_gather(tbuf, [beam * _T + t])
                plsc.store_scatter(obuf_t, [lanes * (_T + 1) + t], gt)
            plsc.store_scatter(obuf_t, [lanes * (_T + 1) + _T], newtok)
            obuf_s[...] = sd
            pltpu.sync_copy(
                obuf_t, otok_hbm.at[pl.ds(b * _K * (_T + 1), _K * (_T + 1))])
            pltpu.sync_copy(obuf_s, osc_hbm.at[pl.ds(b * _K, _K)])
            return 0

        lax.fori_loop(0, _BPW, batch_body, 0)

    return run(joint_flat, tokens_flat)


def kernel(current_log_probs_extended, valid_mask, beam_tokens, beam_scores, k):
    del k
    l5 = current_log_probs_extended.reshape(
        _BS, 2, 8, 256, 128).transpose(0, 1, 3, 2, 4)
    m5 = valid_mask.reshape(_BS, 2, 8, 256, 128).transpose(0, 1, 3, 2, 4)
    s5 = beam_scores.reshape(_BS, 2, 8)[:, :, None, :, None]
    joint = jnp.where(m5, l5 + s5, _NEG).reshape(-1)
    tk = beam_tokens.astype(jnp.int32).reshape(-1)
    otok, osc = _sc_topk(joint, tk)
    new_tokens = otok.reshape(_BS, _K, _T + 1).astype(beam_tokens.dtype)
    return (new_tokens, osc.reshape(_BS, _K))


# speedup vs baseline: 3.5326x; 3.5326x over previous
import functools

import jax
import jax.numpy as jnp
from jax import lax
from jax.experimental import pallas as pl
from jax.experimental.pallas import tpu as pltpu
from jax.experimental.pallas import tpu_sc as plsc

_BS, _K, _N, _T = 64, 16, 32768, 3
_L = 16
_NW = 32
_NSPLIT = 2
_BPC = _BS // _NSPLIT
_BPW = _BPC // _NW
_GROUP = 32
_CH = 32768
_NCH = _K * _N // _CH
_NG = _CH // (_GROUP * _L)
_NEG = float("-inf")


def _iota():
    return lax.iota(jnp.int32, _L)


def _sc_topk(joint_flat, tokens_flat):
    mesh = plsc.VectorSubcoreMesh(core_axis_name="c", subcore_axis_name="s")

    @functools.partial(
        pl.kernel,
        out_type=(
            jax.ShapeDtypeStruct((_BPC * _K * (_T + 1),), jnp.int32),
            jax.ShapeDtypeStruct((_BPC * _K,), jnp.float32),
        ),
        mesh=mesh,
        compiler_params=pltpu.CompilerParams(needs_layout_passes=False),
        scratch_types=[
            pltpu.VMEM((2 * _CH,), jnp.float32),
            pltpu.VMEM((_K * _T,), jnp.int32),
            pltpu.VMEM((_K * (_T + 1),), jnp.int32),
            pltpu.VMEM((_K,), jnp.float32),
            pltpu.SemaphoreType.DMA,
        ],
    )
    def run(jt_hbm, tok_hbm, otok_hbm, osc_hbm, lbuf, tbuf,
            obuf_t, obuf_s, sem):
        wid = lax.axis_index("s") * 2 + lax.axis_index("c")

        def batch_body(bi, _):
            b = wid * _BPW + bi
            base = b * _K * _N
            pltpu.sync_copy(tok_hbm.at[pl.ds(b * _K * _T, _K * _T)], tbuf)
            pltpu.async_copy(jt_hbm.at[pl.ds(base, _CH)],
                             lbuf.at[pl.ds(0, _CH)], sem)

            def chunk_body(ch, carry):
                cur = ch & 1
                loff = cur * _CH
                pltpu.make_async_copy(
                    jt_hbm.at[pl.ds(base + ch * _CH, _CH)],
                    lbuf.at[pl.ds(loff, _CH)], sem).wait()

                @pl.when(ch + 1 < _NCH)
                def _():
                    nxt = (ch + 1) & 1
                    pltpu.async_copy(
                        jt_hbm.at[pl.ds(base + (ch + 1) * _CH, _CH)],
                        lbuf.at[pl.ds(nxt * _CH, _CH)], sem)

                def group_body(g, carry2):
                    bv, bix, thr = carry2
                    p0 = g * (_GROUP * _L)
                    accm = lbuf[pl.ds(loff + p0, _L)]
                    for j in range(1, _GROUP):
                        accm = jnp.maximum(
                            accm, lbuf[pl.ds(loff + p0 + j * _L, _L)])

                    def slow(carry3):
                        def slow_body(j, carry4):
                            bv5, bi5, _ = carry4
                            p = p0 + j * _L
                            c = lbuf[pl.ds(loff + p, _L)]
                            ov = ch * _CH + p + _iota()
                            tr = lax.shift_right_logical(ov, 18)
                            ob = ov & (_K * _N // 2 - 1)
                            tc = lax.shift_right_logical(ob, 10)
                            r8 = lax.shift_right_logical(ob, 7) & 7
                            cc = ob & 127
                            idxv = (((tr << 3) + r8) << 15) + (tc << 7) + cc
                            sk, si = plsc.sort_key_val(
                                c, idxv, descending=False)
                            rs = lax.rev(sk, (0,))
                            ri = lax.rev(si, (0,))
                            ge = bv5 >= rs
                            nv = jnp.where(ge, bv5, rs)
                            ni = jnp.where(ge, bi5, ri)
                            nv, ni = plsc.sort_key_val(
                                nv, ni, descending=False)
                            nthr = jnp.broadcast_to(jnp.min(nv), (_L,))
                            return (nv, ni, nthr)
                        return lax.fori_loop(0, _GROUP, slow_body, carry3)

                    return lax.cond(jnp.any(accm > thr), slow,
                                    lambda c3: c3, (bv, bix, thr))
                return lax.fori_loop(0, _NG, group_body, carry)

            init = (jnp.full((_L,), _NEG, jnp.float32),
                    jnp.zeros((_L,), jnp.int32),
                    jnp.full((_L,), _NEG, jnp.float32))
            best_v, best_i, _ = lax.fori_loop(0, _NCH, chunk_body, init)

            sd = lax.rev(best_v, (0,))
            fd = lax.rev(best_i, (0,))
            beam = lax.shift_right_logical(fd, 15)
            newtok = fd & (_N - 1)
            lanes = _iota()
            for t in range(_T):
                gt = plsc.load_gather(tbuf, [beam * _T + t])
                plsc.store_scatter(obuf_t, [lanes * (_T + 1) + t], gt)
            plsc.store_scatter(obuf_t, [lanes * (_T + 1) + _T], newtok)
            obuf_s[...] = sd
            pltpu.sync_copy(
                obuf_t, otok_hbm.at[pl.ds(b * _K * (_T + 1), _K * (_T + 1))])
            pltpu.sync_copy(obuf_s, osc_hbm.at[pl.ds(b * _K, _K)])
            return 0

        lax.fori_loop(0, _BPW, batch_body, 0)

    return run(joint_flat, tokens_flat)


def kernel(current_log_probs_extended, valid_mask, beam_tokens, beam_scores, k):
    del k
    l5 = current_log_probs_extended.reshape(
        _BS, 2, 8, 256, 128).transpose(0, 1, 3, 2, 4)
    m5 = valid_mask.reshape(_BS, 2, 8, 256, 128).transpose(0, 1, 3, 2, 4)
    s5 = beam_scores.reshape(_BS, 2, 8)[:, :, None, :, None]
    tk = beam_tokens.astype(jnp.int32)
    toks, scs = [], []
    for i in range(_NSPLIT):
        sl = slice(i * _BPC, (i + 1) * _BPC)
        joint = jnp.where(m5[sl], l5[sl] + s5[sl], _NEG).reshape(-1)
        otok, osc = _sc_topk(joint, tk[sl].reshape(-1))
        toks.append(otok.reshape(_BPC, _K, _T + 1))
        scs.append(osc.reshape(_BPC, _K))
    new_tokens = jnp.concatenate(toks, 0).astype(beam_tokens.dtype)
    return (new_tokens, jnp.concatenate(scs, 0))


# speedup vs baseline: 3.9995x; 1.1322x over previous
import functools

import jax
import jax.numpy as jnp
from jax import lax
from jax.experimental import pallas as pl
from jax.experimental.pallas import tpu as pltpu
from jax.experimental.pallas import tpu_sc as plsc

_BS, _K, _N, _T = 64, 16, 32768, 3
_L = 16
_NW = 32
_NSPLIT = 1
_BPC = _BS // _NSPLIT
_BPW = _BPC // _NW
_GROUP = 32
_CH = 32768
_NCH = _K * _N // _CH
_NG = _CH // (_GROUP * _L)
_NEG = float("-inf")


def _iota():
    return lax.iota(jnp.int32, _L)


def _sc_topk(joint_flat, tokens_flat):
    mesh = plsc.VectorSubcoreMesh(core_axis_name="c", subcore_axis_name="s")

    @functools.partial(
        pl.kernel,
        out_type=(
            jax.ShapeDtypeStruct((_BPC * _K * (_T + 1),), jnp.int32),
            jax.ShapeDtypeStruct((_BPC * _K,), jnp.float32),
        ),
        mesh=mesh,
        compiler_params=pltpu.CompilerParams(needs_layout_passes=False),
        scratch_types=[
            pltpu.VMEM((2 * _CH,), jnp.float32),
            pltpu.VMEM((_K * _T,), jnp.int32),
            pltpu.VMEM((_K * (_T + 1),), jnp.int32),
            pltpu.VMEM((_K,), jnp.float32),
            pltpu.SemaphoreType.DMA,
        ],
    )
    def run(jt_hbm, tok_hbm, otok_hbm, osc_hbm, lbuf, tbuf,
            obuf_t, obuf_s, sem):
        wid = lax.axis_index("s") * 2 + lax.axis_index("c")

        def batch_body(bi, _):
            b = wid * _BPW + bi
            base = b * _K * _N
            pltpu.sync_copy(tok_hbm.at[pl.ds(b * _K * _T, _K * _T)], tbuf)
            pltpu.async_copy(jt_hbm.at[pl.ds(base, _CH)],
                             lbuf.at[pl.ds(0, _CH)], sem)

            def chunk_body(ch, carry):
                cur = ch & 1
                loff = cur * _CH
                pltpu.make_async_copy(
                    jt_hbm.at[pl.ds(base + ch * _CH, _CH)],
                    lbuf.at[pl.ds(loff, _CH)], sem).wait()

                @pl.when(ch + 1 < _NCH)
                def _():
                    nxt = (ch + 1) & 1
                    pltpu.async_copy(
                        jt_hbm.at[pl.ds(base + (ch + 1) * _CH, _CH)],
                        lbuf.at[pl.ds(nxt * _CH, _CH)], sem)

                def group_body(g, carry2):
                    bv, bix, thr = carry2
                    p0 = g * (_GROUP * _L)
                    accm = lbuf[pl.ds(loff + p0, _L)]
                    for j in range(1, _GROUP):
                        accm = jnp.maximum(
                            accm, lbuf[pl.ds(loff + p0 + j * _L, _L)])

                    def slow(carry3):
                        def sub_body(s, carry4):
                            q0 = p0 + s * (8 * _L)
                            sacc = lbuf[pl.ds(loff + q0, _L)]
                            for j in range(1, 8):
                                sacc = jnp.maximum(
                                    sacc, lbuf[pl.ds(loff + q0 + j * _L, _L)])

                            def merge8(carry5):
                                def slow_body(j, carry6):
                                    bv5, bi5, _ = carry6
                                    p = q0 + j * _L
                                    c = lbuf[pl.ds(loff + p, _L)]
                                    ov = ch * _CH + p + _iota()
                                    tr = lax.shift_right_logical(ov, 18)
                                    ob = ov & (_K * _N // 2 - 1)
                                    tc = lax.shift_right_logical(ob, 10)
                                    r8 = lax.shift_right_logical(ob, 7) & 7
                                    cc = ob & 127
                                    idxv = ((((tr << 3) + r8) << 15)
                                            + (tc << 7) + cc)
                                    sk, si = plsc.sort_key_val(
                                        c, idxv, descending=False)
                                    rs = lax.rev(sk, (0,))
                                    ri = lax.rev(si, (0,))
                                    ge = bv5 >= rs
                                    nv = jnp.where(ge, bv5, rs)
                                    ni = jnp.where(ge, bi5, ri)
                                    nv, ni = plsc.sort_key_val(
                                        nv, ni, descending=False)
                                    nthr = jnp.broadcast_to(
                                        jnp.min(nv), (_L,))
                                    return (nv, ni, nthr)
                                return lax.fori_loop(0, 8, slow_body, carry5)

                            return lax.cond(jnp.any(sacc > carry4[2]),
                                            merge8, lambda c5: c5, carry4)
                        return lax.fori_loop(0, _GROUP // 8, sub_body, carry3)

                    return lax.cond(jnp.any(accm > thr), slow,
                                    lambda c3: c3, (bv, bix, thr))
                return lax.fori_loop(0, _NG, group_body, carry)

            init = (jnp.full((_L,), _NEG, jnp.float32),
                    jnp.zeros((_L,), jnp.int32),
                    jnp.full((_L,), _NEG, jnp.float32))
            best_v, best_i, _ = lax.fori_loop(0, _NCH, chunk_body, init)

            sd = lax.rev(best_v, (0,))
            fd = lax.rev(best_i, (0,))
            beam = lax.shift_right_logical(fd, 15)
            newtok = fd & (_N - 1)
            lanes = _iota()
            for t in range(_T):
                gt = plsc.load_gather(tbuf, [beam * _T + t])
                plsc.store_scatter(obuf_t, [lanes * (_T + 1) + t], gt)
            plsc.store_scatter(obuf_t, [lanes * (_T + 1) + _T], newtok)
            obuf_s[...] = sd
            pltpu.sync_copy(
                obuf_t, otok_hbm.at[pl.ds(b * _K * (_T + 1), _K * (_T + 1))])
            pltpu.sync_copy(obuf_s, osc_hbm.at[pl.ds(b * _K, _K)])
            return 0

        lax.fori_loop(0, _BPW, batch_body, 0)

    return run(joint_flat, tokens_flat)


def kernel(current_log_probs_extended, valid_mask, beam_tokens, beam_scores, k):
    del k
    l5 = current_log_probs_extended.reshape(
        _BS, 2, 8, 256, 128).transpose(0, 1, 3, 2, 4)
    m5 = valid_mask.reshape(_BS, 2, 8, 256, 128).transpose(0, 1, 3, 2, 4)
    s5 = beam_scores.reshape(_BS, 2, 8)[:, :, None, :, None]
    tk = beam_tokens.astype(jnp.int32)
    toks, scs = [], []
    for i in range(_NSPLIT):
        sl = slice(i * _BPC, (i + 1) * _BPC)
        joint = jnp.where(m5[sl], l5[sl] + s5[sl], _NEG).reshape(-1)
        otok, osc = _sc_topk(joint, tk[sl].reshape(-1))
        toks.append(otok.reshape(_BPC, _K, _T + 1))
        scs.append(osc.reshape(_BPC, _K))
    new_tokens = jnp.concatenate(toks, 0).astype(beam_tokens.dtype)
    return (new_tokens, jnp.concatenate(scs, 0))


# speedup vs baseline: 4.0983x; 1.0247x over previous
import functools

import jax
import jax.numpy as jnp
from jax import lax
from jax.experimental import pallas as pl
from jax.experimental.pallas import tpu as pltpu
from jax.experimental.pallas import tpu_sc as plsc

_BS, _K, _N, _T = 64, 16, 32768, 3
_L = 16
_NW = 32
_NSPLIT = 1
_BPC = _BS // _NSPLIT
_BPW = _BPC // _NW
_GROUP = 64
_CH = 32768
_NCH = _K * _N // _CH
_NG = _CH // (_GROUP * _L)
_NEG = float("-inf")


def _iota():
    return lax.iota(jnp.int32, _L)


def _sc_topk(joint_flat, tokens_flat):
    mesh = plsc.VectorSubcoreMesh(core_axis_name="c", subcore_axis_name="s")

    @functools.partial(
        pl.kernel,
        out_type=(
            jax.ShapeDtypeStruct((_BPC * _K * (_T + 1),), jnp.int32),
            jax.ShapeDtypeStruct((_BPC * _K,), jnp.float32),
        ),
        mesh=mesh,
        compiler_params=pltpu.CompilerParams(needs_layout_passes=False),
        scratch_types=[
            pltpu.VMEM((2 * _CH,), jnp.float32),
            pltpu.VMEM((_K * _T,), jnp.int32),
            pltpu.VMEM((_K * (_T + 1),), jnp.int32),
            pltpu.VMEM((_K,), jnp.float32),
            pltpu.SemaphoreType.DMA,
        ],
    )
    def run(jt_hbm, tok_hbm, otok_hbm, osc_hbm, lbuf, tbuf,
            obuf_t, obuf_s, sem):
        wid = lax.axis_index("s") * 2 + lax.axis_index("c")

        def batch_body(bi, _):
            b = wid * _BPW + bi
            base = b * _K * _N
            pltpu.sync_copy(tok_hbm.at[pl.ds(b * _K * _T, _K * _T)], tbuf)
            pltpu.async_copy(jt_hbm.at[pl.ds(base, _CH)],
                             lbuf.at[pl.ds(0, _CH)], sem)

            def chunk_body(ch, carry):
                cur = ch & 1
                loff = cur * _CH
                pltpu.make_async_copy(
                    jt_hbm.at[pl.ds(base + ch * _CH, _CH)],
                    lbuf.at[pl.ds(loff, _CH)], sem).wait()

                @pl.when(ch + 1 < _NCH)
                def _():
                    nxt = (ch + 1) & 1
                    pltpu.async_copy(
                        jt_hbm.at[pl.ds(base + (ch + 1) * _CH, _CH)],
                        lbuf.at[pl.ds(nxt * _CH, _CH)], sem)

                def group_body(g, carry2):
                    bv, bix, thr = carry2
                    p0 = g * (_GROUP * _L)
                    accm = lbuf[pl.ds(loff + p0, _L)]
                    for j in range(1, _GROUP):
                        accm = jnp.maximum(
                            accm, lbuf[pl.ds(loff + p0 + j * _L, _L)])

                    def slow(carry3):
                        def sub_body(s, carry4):
                            q0 = p0 + s * (8 * _L)
                            sacc = lbuf[pl.ds(loff + q0, _L)]
                            for j in range(1, 8):
                                sacc = jnp.maximum(
                                    sacc, lbuf[pl.ds(loff + q0 + j * _L, _L)])

                            def merge8(carry5):
                                def slow_body(j, carry6):
                                    bv5, bi5, _ = carry6
                                    p = q0 + j * _L
                                    c = lbuf[pl.ds(loff + p, _L)]
                                    ov = ch * _CH + p + _iota()
                                    tr = lax.shift_right_logical(ov, 18)
                                    ob = ov & (_K * _N // 2 - 1)
                                    tc = lax.shift_right_logical(ob, 10)
                                    r8 = lax.shift_right_logical(ob, 7) & 7
                                    cc = ob & 127
                                    idxv = ((((tr << 3) + r8) << 15)
                                            + (tc << 7) + cc)
                                    sk, si = plsc.sort_key_val(
                                        c, idxv, descending=False)
                                    rs = lax.rev(sk, (0,))
                                    ri = lax.rev(si, (0,))
                                    ge = bv5 >= rs
                                    nv = jnp.where(ge, bv5, rs)
                                    ni = jnp.where(ge, bi5, ri)
                                    nv, ni = plsc.sort_key_val(
                                        nv, ni, descending=False)
                                    nthr = jnp.broadcast_to(
                                        jnp.min(nv), (_L,))
                                    return (nv, ni, nthr)
                                return lax.fori_loop(0, 8, slow_body, carry5)

                            return lax.cond(jnp.any(sacc > carry4[2]),
                                            merge8, lambda c5: c5, carry4)
                        return lax.fori_loop(0, _GROUP // 8, sub_body, carry3)

                    return lax.cond(jnp.any(accm > thr), slow,
                                    lambda c3: c3, (bv, bix, thr))
                return lax.fori_loop(0, _NG, group_body, carry)

            init = (jnp.full((_L,), _NEG, jnp.float32),
                    jnp.zeros((_L,), jnp.int32),
                    jnp.full((_L,), _NEG, jnp.float32))
            best_v, best_i, _ = lax.fori_loop(0, _NCH, chunk_body, init)

            sd = lax.rev(best_v, (0,))
            fd = lax.rev(best_i, (0,))
            beam = lax.shift_right_logical(fd, 15)
            newtok = fd & (_N - 1)
            lanes = _iota()
            for t in range(_T):
                gt = plsc.load_gather(tbuf, [beam * _T + t])
                plsc.store_scatter(obuf_t, [lanes * (_T + 1) + t], gt)
            plsc.store_scatter(obuf_t, [lanes * (_T + 1) + _T], newtok)
            obuf_s[...] = sd
            pltpu.sync_copy(
                obuf_t, otok_hbm.at[pl.ds(b * _K * (_T + 1), _K * (_T + 1))])
            pltpu.sync_copy(obuf_s, osc_hbm.at[pl.ds(b * _K, _K)])
            return 0

        lax.fori_loop(0, _BPW, batch_body, 0)

    return run(joint_flat, tokens_flat)


def kernel(current_log_probs_extended, valid_mask, beam_tokens, beam_scores, k):
    del k
    l5 = current_log_probs_extended.reshape(
        _BS, 2, 8, 256, 128).transpose(0, 1, 3, 2, 4)
    m5 = valid_mask.reshape(_BS, 2, 8, 256, 128).transpose(0, 1, 3, 2, 4)
    s5 = beam_scores.reshape(_BS, 2, 8)[:, :, None, :, None]
    tk = beam_tokens.astype(jnp.int32)
    toks, scs = [], []
    for i in range(_NSPLIT):
        sl = slice(i * _BPC, (i + 1) * _BPC)
        joint = jnp.where(m5[sl], l5[sl] + s5[sl], _NEG).reshape(-1)
        otok, osc = _sc_topk(joint, tk[sl].reshape(-1))
        toks.append(otok.reshape(_BPC, _K, _T + 1))
        scs.append(osc.reshape(_BPC, _K))
    new_tokens = jnp.concatenate(toks, 0).astype(beam_tokens.dtype)
    return (new_tokens, jnp.concatenate(scs, 0))
